# baseline (speedup 1.0000x reference)
import functools

import jax
import jax.numpy as jnp
from jax import lax
from jax.experimental import pallas as pl
from jax.experimental.pallas import tpu as pltpu
from jax.experimental.pallas import tpu_sc as plsc

N_NODES = 10000
H = 128
NUM_BASES = 16

NUM_WORKERS = 32
CHUNK = 128
BLK_E = 1024
NCH = 4
IB = 4
BLK_N = 1000
N_PAD = 10240
ROWS_PER_TILE = N_PAD // 16


def _sc_mesh():
    return plsc.VectorSubcoreMesh(core_axis_name="core", subcore_axis_name="subcore")


def _sc_gather(h, src_chunks, epad):
    nchunks = epad // CHUNK
    per_tile = nchunks // NUM_WORKERS

    @functools.partial(
        pl.kernel,
        out_type=jax.ShapeDtypeStruct((epad, H), jnp.float32),
        mesh=_sc_mesh(),
        scratch_types=[
            pltpu.VMEM((CHUNK,), jnp.int32),
            pltpu.VMEM((CHUNK, H), jnp.float32),
        ],
    )
    def gather_kernel(h_hbm, src_hbm, hs_hbm, src_v, hs_v):
        cid = lax.axis_index("core")
        sid = lax.axis_index("subcore")
        wid = sid * 2 + cid

        @pl.loop(0, per_tile)
        def _(j):
            base = (wid * per_tile + j) * CHUNK
            pltpu.sync_copy(src_hbm.at[pl.ds(base, CHUNK)], src_v)
            pltpu.sync_copy(h_hbm.at[src_v], hs_v)
            pltpu.sync_copy(hs_v, hs_hbm.at[pl.ds(base, CHUNK)])

    return gather_kernel(h, src_chunks)


def _tc_transform(hs, typ_col, norm_col, coeff_exp, basis_stack, epad,
                  hs_off_blk=0):
    nblk = epad // BLK_E
    num_rel = coeff_exp.shape[0]

    def body(hs_ref, typ_ref, nrm_ref, coeff_ref, basis_ref, out_ref):
        rel_iota = lax.broadcasted_iota(jnp.int32, (BLK_E, num_rel), 1)
        onehot = ((typ_ref[...] == rel_iota).astype(jnp.float32)
                  * nrm_ref[...]).astype(jnp.bfloat16)
        ce_exp = jnp.dot(onehot, coeff_ref[...],
                         preferred_element_type=jnp.float32
                         ).astype(jnp.bfloat16)
        hs_b = hs_ref[...].astype(jnp.bfloat16)
        hs_t = jnp.concatenate([hs_b] * NUM_BASES, axis=1)
        x = hs_t * ce_exp
        out_ref[...] = jnp.dot(x, basis_ref[...],
                               preferred_element_type=jnp.float32)

    return pl.pallas_call(
        body,
        grid=(nblk,),
        in_specs=[
            pl.BlockSpec((BLK_E, H), lambda i, o=hs_off_blk: (i + o, 0)),
            pl.BlockSpec((BLK_E, 1), lambda i: (i, 0)),
            pl.BlockSpec((BLK_E, 1), lambda i: (i, 0)),
            pl.BlockSpec((num_rel, NUM_BASES * H), lambda i: (0, 0)),
            pl.BlockSpec((NUM_BASES * H, H), lambda i: (0, 0)),
        ],
        out_specs=pl.BlockSpec((BLK_E, H), lambda i: (i, 0)),
        out_shape=jax.ShapeDtypeStruct((epad, H), jnp.float32),
    )(hs, typ_col, norm_col, coeff_exp, basis_stack)


def _sc_scatter(msg, dst_chunks, zeros_rows, epad):
    nchunks = epad // CHUNK
    per_tile = nchunks // NUM_WORKERS

    @functools.partial(
        pl.kernel,
        out_type=jax.ShapeDtypeStruct((2, N_PAD, H), jnp.float32),
        mesh=_sc_mesh(),
        scratch_types=[
            pltpu.VMEM_SHARED((N_PAD, H), jnp.float32),
            pltpu.VMEM((CHUNK, H), jnp.float32),
            pltpu.VMEM((1, CHUNK), jnp.int32),
        ],
    )
    def scatter_kernel(msg_hbm, dst_hbm, zero_hbm, out_hbm,
                       agg_sh, msg_v, dst_v):
        cid = lax.axis_index("core")
        sid = lax.axis_index("subcore")
        wid = sid * 2 + cid

        pltpu.sync_copy(zero_hbm, agg_sh.at[pl.ds(sid * ROWS_PER_TILE, ROWS_PER_TILE)])
        plsc.subcore_barrier()

        @pl.loop(0, per_tile)
        def _(j):
            ch = wid * per_tile + j
            base = ch * CHUNK
            pltpu.sync_copy(msg_hbm.at[pl.ds(base, CHUNK)], msg_v)
            pltpu.sync_copy(dst_hbm.at[ch], dst_v.at[0])
            pltpu.sync_copy(msg_v, agg_sh.at[dst_v.at[0]], add=True)

        plsc.subcore_barrier()
        pltpu.sync_copy(
            agg_sh.at[pl.ds(sid * ROWS_PER_TILE, ROWS_PER_TILE)],
            out_hbm.at[cid, pl.ds(sid * ROWS_PER_TILE, ROWS_PER_TILE)])

    return scatter_kernel(msg, dst_chunks, zeros_rows)


def _sc_scatter_gather(msg, dst_chunks, h, src_next, zeros_rows, epad):
    nchunks = epad // CHUNK
    per_tile = nchunks // NUM_WORKERS

    @functools.partial(
        pl.kernel,
        out_type=(
            jax.ShapeDtypeStruct((2, N_PAD, H), jnp.float32),
            jax.ShapeDtypeStruct((epad, H), jnp.float32),
        ),
        mesh=_sc_mesh(),
        scratch_types=[
            pltpu.VMEM_SHARED((N_PAD, H), jnp.float32),
            pltpu.VMEM((CHUNK, H), jnp.float32),
            pltpu.VMEM((1, CHUNK), jnp.int32),
            pltpu.VMEM((CHUNK,), jnp.int32),
            pltpu.VMEM((CHUNK, H), jnp.float32),
        ],
    )
    def sg_kernel(msg_hbm, dst_hbm, h_hbm, src_hbm, zero_hbm,
                  agg_hbm, hs_hbm, agg_sh, msg_v, dst_v, src_v, hs_v):
        cid = lax.axis_index("core")
        sid = lax.axis_index("subcore")
        wid = sid * 2 + cid

        pltpu.sync_copy(zero_hbm,
                        agg_sh.at[pl.ds(sid * ROWS_PER_TILE, ROWS_PER_TILE)])

        @pl.loop(0, per_tile)
        def _(j):
            base = (wid * per_tile + j) * CHUNK
            pltpu.sync_copy(src_hbm.at[pl.ds(base, CHUNK)], src_v)
            pltpu.sync_copy(h_hbm.at[src_v], hs_v)
            pltpu.sync_copy(hs_v, hs_hbm.at[pl.ds(base, CHUNK)])

        plsc.subcore_barrier()

        @pl.loop(0, per_tile)
        def _(j):
            ch = wid * per_tile + j
            pltpu.sync_copy(msg_hbm.at[pl.ds(ch * CHUNK, CHUNK)], msg_v)
            pltpu.sync_copy(dst_hbm.at[ch], dst_v.at[0])
            pltpu.sync_copy(msg_v, agg_sh.at[dst_v.at[0]], add=True)

        plsc.subcore_barrier()
        pltpu.sync_copy(
            agg_sh.at[pl.ds(sid * ROWS_PER_TILE, ROWS_PER_TILE)],
            agg_hbm.at[cid, pl.ds(sid * ROWS_PER_TILE, ROWS_PER_TILE)])

    return sg_kernel(msg, dst_chunks, h, src_next, zeros_rows)


def _tc_final(aggs, h, w_self, bias_row):
    nblk = N_NODES // BLK_N
    naggs = len(aggs)

    def body(*refs):
        agg_refs = refs[:naggs]
        h_ref, w_ref, b_ref, out_ref = refs[naggs:]
        acc = jnp.dot(h_ref[...], w_ref[...],
                      preferred_element_type=jnp.float32) + b_ref[...]
        for agg_ref in agg_refs:
            acc = acc + agg_ref[0] + agg_ref[1]
        out_ref[...] = acc

    return pl.pallas_call(
        body,
        grid=(nblk,),
        in_specs=(
            [pl.BlockSpec((2, BLK_N, H), lambda i: (0, i, 0))] * naggs
            + [
                pl.BlockSpec((BLK_N, H), lambda i: (i, 0)),
                pl.BlockSpec((H, H), lambda i: (0, 0)),
                pl.BlockSpec((1, H), lambda i: (0, 0)),
            ]
        ),
        out_specs=pl.BlockSpec((BLK_N, H), lambda i: (i, 0)),
        out_shape=jax.ShapeDtypeStruct((N_NODES, H), jnp.float32),
    )(*aggs, h, w_self, bias_row)


def kernel(node_id, edge_index, edge_type, edge_norm, emb_table, basis, coeff,
           w_self, bias):
    del node_id
    h = emb_table
    n_edges = edge_index.shape[1]

    grain = CHUNK * NUM_WORKERS * NCH
    epad = ((n_edges + grain - 1) // grain) * grain
    pad = epad - n_edges

    src = jnp.pad(edge_index[0].astype(jnp.int32), (0, pad))
    dst = jnp.pad(edge_index[1].astype(jnp.int32), (0, pad))
    typ = jnp.pad(edge_type.astype(jnp.int32), (0, pad))
    norm = jnp.pad(edge_norm.astype(jnp.float32), (0, pad))

    nchunks = epad // CHUNK
    src_chunks = src.reshape(nchunks, CHUNK)
    dst_chunks = dst.reshape(nchunks, CHUNK)
    typ_col = typ.reshape(epad, 1)
    norm_col = norm.reshape(epad, 1)
    esub = epad // NCH
    csub = nchunks // NCH

    basis_stack = basis.reshape(NUM_BASES * H, H).astype(jnp.bfloat16)
    coeff_exp = jnp.broadcast_to(
        coeff.astype(jnp.bfloat16)[:, :, None],
        (coeff.shape[0], NUM_BASES, H)).reshape(coeff.shape[0], NUM_BASES * H)
    bias_row = bias.reshape(1, H)
    zeros_rows = jnp.zeros((ROWS_PER_TILE, H), jnp.float32)

    def sub(col, c):
        return col[c * esub:(c + 1) * esub]

    b_per = esub // BLK_E
    hs01 = _sc_gather(h, src[:2 * esub], 2 * esub)
    msg0 = _tc_transform(hs01, sub(typ_col, 0), sub(norm_col, 0),
                         coeff_exp, basis_stack, esub, 0)
    msg1 = _tc_transform(hs01, sub(typ_col, 1), sub(norm_col, 1),
                         coeff_exp, basis_stack, esub, b_per)
    agg0, hs2 = _sc_scatter_gather(msg0, dst_chunks[0:csub], h,
                                   src[2 * esub:3 * esub], zeros_rows, esub)
    msg2 = _tc_transform(hs2, sub(typ_col, 2), sub(norm_col, 2),
                         coeff_exp, basis_stack, esub, 0)
    agg1, hs3 = _sc_scatter_gather(msg1, dst_chunks[csub:2 * csub], h,
                                   src[3 * esub:4 * esub], zeros_rows, esub)
    msg3 = _tc_transform(hs3, sub(typ_col, 3), sub(norm_col, 3),
                         coeff_exp, basis_stack, esub, 0)
    agg2 = _sc_scatter(msg2, dst_chunks[2 * csub:3 * csub], zeros_rows, esub)
    agg3 = _sc_scatter(msg3, dst_chunks[3 * csub:4 * csub], zeros_rows, esub)
    out = _tc_final([a[:, :N_NODES] for a in (agg0, agg1, agg2, agg3)],
                    h, w_self, bias_row)
    return out


# speedup vs baseline: 1.1719x; 1.1719x over previous
import functools

import jax
import jax.numpy as jnp
from jax import lax
from jax.experimental import pallas as pl
from jax.experimental.pallas import tpu as pltpu
from jax.experimental.pallas import tpu_sc as plsc

N_NODES = 10000
H = 128
NUM_BASES = 16

NUM_WORKERS = 32
CHUNK = 128
BLK_E = 1024
NCH = 5
BLK_N = 1000
N_PAD = 10240
ROWS_PER_TILE = N_PAD // 16


def _sc_mesh():
    return plsc.VectorSubcoreMesh(core_axis_name="core", subcore_axis_name="subcore")


def _sc_gather(h, src_chunks, epad):
    nchunks = epad // CHUNK
    per_tile = nchunks // NUM_WORKERS

    @functools.partial(
        pl.kernel,
        out_type=jax.ShapeDtypeStruct((epad, H), jnp.float32),
        mesh=_sc_mesh(),
        scratch_types=[
            pltpu.VMEM((CHUNK,), jnp.int32),
            pltpu.VMEM((CHUNK, H), jnp.float32),
        ],
    )
    def gather_kernel(h_hbm, src_hbm, hs_hbm, src_v, hs_v):
        cid = lax.axis_index("core")
        sid = lax.axis_index("subcore")
        wid = sid * 2 + cid

        @pl.loop(0, per_tile)
        def _(j):
            base = (wid * per_tile + j) * CHUNK
            pltpu.sync_copy(src_hbm.at[pl.ds(base, CHUNK)], src_v)
            pltpu.sync_copy(h_hbm.at[src_v], hs_v)
            pltpu.sync_copy(hs_v, hs_hbm.at[pl.ds(base, CHUNK)])

    return gather_kernel(h, src_chunks)


def _tc_transform(hs, typ_col, norm_col, coeff_exp, basis_stack, epad,
                  hs_off_blk=0):
    nblk = epad // BLK_E
    num_rel = coeff_exp.shape[0]

    def body(hs_ref, typ_ref, nrm_ref, coeff_ref, basis_ref, out_ref):
        rel_iota = lax.broadcasted_iota(jnp.int32, (BLK_E, num_rel), 1)
        onehot = ((typ_ref[...] == rel_iota).astype(jnp.float32)
                  * nrm_ref[...]).astype(jnp.bfloat16)
        ce_exp = jnp.dot(onehot, coeff_ref[...],
                         preferred_element_type=jnp.float32
                         ).astype(jnp.bfloat16)
        hs_b = hs_ref[...].astype(jnp.bfloat16)
        hs_t = jnp.concatenate([hs_b] * NUM_BASES, axis=1)
        x = hs_t * ce_exp
        out_ref[...] = jnp.dot(x, basis_ref[...],
                               preferred_element_type=jnp.float32)

    return pl.pallas_call(
        body,
        grid=(nblk,),
        in_specs=[
            pl.BlockSpec((BLK_E, H), lambda i, o=hs_off_blk: (i + o, 0)),
            pl.BlockSpec((BLK_E, 1), lambda i: (i, 0)),
            pl.BlockSpec((BLK_E, 1), lambda i: (i, 0)),
            pl.BlockSpec((num_rel, NUM_BASES * H), lambda i: (0, 0)),
            pl.BlockSpec((NUM_BASES * H, H), lambda i: (0, 0)),
        ],
        out_specs=pl.BlockSpec((BLK_E, H), lambda i: (i, 0)),
        out_shape=jax.ShapeDtypeStruct((epad, H), jnp.float32),
    )(hs, typ_col, norm_col, coeff_exp, basis_stack)


def _sc_scatter(msg, dst_chunks, zeros_rows, epad):
    nchunks = epad // CHUNK
    per_tile = nchunks // NUM_WORKERS

    @functools.partial(
        pl.kernel,
        out_type=jax.ShapeDtypeStruct((2, N_PAD, H), jnp.float32),
        mesh=_sc_mesh(),
        scratch_types=[
            pltpu.VMEM_SHARED((N_PAD, H), jnp.float32),
            pltpu.VMEM((CHUNK, H), jnp.float32),
            pltpu.VMEM((1, CHUNK), jnp.int32),
        ],
    )
    def scatter_kernel(msg_hbm, dst_hbm, zero_hbm, out_hbm,
                       agg_sh, msg_v, dst_v):
        cid = lax.axis_index("core")
        sid = lax.axis_index("subcore")
        wid = sid * 2 + cid

        pltpu.sync_copy(zero_hbm, agg_sh.at[pl.ds(sid * ROWS_PER_TILE, ROWS_PER_TILE)])
        plsc.subcore_barrier()

        @pl.loop(0, per_tile)
        def _(j):
            ch = wid * per_tile + j
            base = ch * CHUNK
            pltpu.sync_copy(msg_hbm.at[pl.ds(base, CHUNK)], msg_v)
            pltpu.sync_copy(dst_hbm.at[ch], dst_v.at[0])
            pltpu.sync_copy(msg_v, agg_sh.at[dst_v.at[0]], add=True)

        plsc.subcore_barrier()
        pltpu.sync_copy(
            agg_sh.at[pl.ds(sid * ROWS_PER_TILE, ROWS_PER_TILE)],
            out_hbm.at[cid, pl.ds(sid * ROWS_PER_TILE, ROWS_PER_TILE)])

    return scatter_kernel(msg, dst_chunks, zeros_rows)


def _tc_final(aggs, h, w_self, bias_row):
    nblk = N_NODES // BLK_N
    naggs = len(aggs)

    def body(*refs):
        agg_refs = refs[:naggs]
        h_ref, w_ref, b_ref, out_ref = refs[naggs:]
        acc = jnp.dot(h_ref[...], w_ref[...],
                      preferred_element_type=jnp.float32) + b_ref[...]
        for agg_ref in agg_refs:
            acc = acc + agg_ref[0] + agg_ref[1]
        out_ref[...] = acc

    return pl.pallas_call(
        body,
        grid=(nblk,),
        in_specs=(
            [pl.BlockSpec((2, BLK_N, H), lambda i: (0, i, 0))] * naggs
            + [
                pl.BlockSpec((BLK_N, H), lambda i: (i, 0)),
                pl.BlockSpec((H, H), lambda i: (0, 0)),
                pl.BlockSpec((1, H), lambda i: (0, 0)),
            ]
        ),
        out_specs=pl.BlockSpec((BLK_N, H), lambda i: (i, 0)),
        out_shape=jax.ShapeDtypeStruct((N_NODES, H), jnp.float32),
    )(*aggs, h, w_self, bias_row)


def kernel(node_id, edge_index, edge_type, edge_norm, emb_table, basis, coeff,
           w_self, bias):
    del node_id
    h = emb_table
    n_edges = edge_index.shape[1]

    grain = CHUNK * NUM_WORKERS * NCH
    epad = ((n_edges + grain - 1) // grain) * grain
    pad = epad - n_edges

    src = jnp.pad(edge_index[0].astype(jnp.int32), (0, pad))
    dst = jnp.pad(edge_index[1].astype(jnp.int32), (0, pad))
    typ = jnp.pad(edge_type.astype(jnp.int32), (0, pad))
    norm = jnp.pad(edge_norm.astype(jnp.float32), (0, pad))

    nchunks = epad // CHUNK
    src_chunks = src.reshape(nchunks, CHUNK)
    dst_chunks = dst.reshape(nchunks, CHUNK)
    typ_col = typ.reshape(epad, 1)
    norm_col = norm.reshape(epad, 1)
    esub = epad // NCH
    csub = nchunks // NCH

    basis_stack = basis.reshape(NUM_BASES * H, H).astype(jnp.bfloat16)
    coeff_exp = jnp.broadcast_to(
        coeff.astype(jnp.bfloat16)[:, :, None],
        (coeff.shape[0], NUM_BASES, H)).reshape(coeff.shape[0], NUM_BASES * H)
    bias_row = bias.reshape(1, H)
    zeros_rows = jnp.zeros((ROWS_PER_TILE, H), jnp.float32)

    aggs = []
    for c in range(NCH):
        hs_c = _sc_gather(h, src[c * esub:(c + 1) * esub], esub)
        msg_c = _tc_transform(hs_c,
                              typ_col[c * esub:(c + 1) * esub],
                              norm_col[c * esub:(c + 1) * esub],
                              coeff_exp, basis_stack, esub)
        aggs.append(_sc_scatter(msg_c,
                                dst_chunks[c * csub:(c + 1) * csub],
                                zeros_rows, esub))
    out = _tc_final([a[:, :N_NODES] for a in aggs], h, w_self, bias_row)
    return out


# speedup vs baseline: 1.1842x; 1.0105x over previous
import functools

import jax
import jax.numpy as jnp
from jax import lax
from jax.experimental import pallas as pl
from jax.experimental.pallas import tpu as pltpu
from jax.experimental.pallas import tpu_sc as plsc

N_NODES = 10000
H = 128
NUM_BASES = 16

NUM_WORKERS = 32
CHUNK = 128
BLK_E = 1024
NCH = 4
BLK_N = 1000
N_PAD = 10240
ROWS_PER_TILE = N_PAD // 16


def _sc_mesh():
    return plsc.VectorSubcoreMesh(core_axis_name="core", subcore_axis_name="subcore")


def _sc_gather(h, src_chunks, epad):
    nchunks = epad // CHUNK
    per_tile = nchunks // NUM_WORKERS

    @functools.partial(
        pl.kernel,
        out_type=jax.ShapeDtypeStruct((epad, H), jnp.float32),
        mesh=_sc_mesh(),
        scratch_types=[
            pltpu.VMEM((CHUNK,), jnp.int32),
            pltpu.VMEM((CHUNK, H), jnp.float32),
        ],
    )
    def gather_kernel(h_hbm, src_hbm, hs_hbm, src_v, hs_v):
        cid = lax.axis_index("core")
        sid = lax.axis_index("subcore")
        wid = sid * 2 + cid

        @pl.loop(0, per_tile)
        def _(j):
            base = (wid * per_tile + j) * CHUNK
            pltpu.sync_copy(src_hbm.at[pl.ds(base, CHUNK)], src_v)
            pltpu.sync_copy(h_hbm.at[src_v], hs_v)
            pltpu.sync_copy(hs_v, hs_hbm.at[pl.ds(base, CHUNK)])

    return gather_kernel(h, src_chunks)


def _tc_transform(hs, typ_col, norm_col, coeff_exp, basis_stack, epad,
                  hs_off_blk=0):
    nblk = epad // BLK_E
    num_rel = coeff_exp.shape[0]

    def body(hs_ref, typ_ref, nrm_ref, coeff_ref, basis_ref, out_ref):
        rel_iota = lax.broadcasted_iota(jnp.int32, (BLK_E, num_rel), 1)
        onehot = ((typ_ref[...] == rel_iota).astype(jnp.float32)
                  * nrm_ref[...]).astype(jnp.bfloat16)
        ce_exp = jnp.dot(onehot, coeff_ref[...],
                         preferred_element_type=jnp.float32
                         ).astype(jnp.bfloat16)
        hs_b = hs_ref[...].astype(jnp.bfloat16)
        hs_t = jnp.concatenate([hs_b] * NUM_BASES, axis=1)
        x = hs_t * ce_exp
        out_ref[...] = jnp.dot(x, basis_ref[...],
                               preferred_element_type=jnp.float32)

    return pl.pallas_call(
        body,
        grid=(nblk,),
        in_specs=[
            pl.BlockSpec((BLK_E, H), lambda i, o=hs_off_blk: (i + o, 0)),
            pl.BlockSpec((BLK_E, 1), lambda i: (i, 0)),
            pl.BlockSpec((BLK_E, 1), lambda i: (i, 0)),
            pl.BlockSpec((num_rel, NUM_BASES * H), lambda i: (0, 0)),
            pl.BlockSpec((NUM_BASES * H, H), lambda i: (0, 0)),
        ],
        out_specs=pl.BlockSpec((BLK_E, H), lambda i: (i, 0)),
        out_shape=jax.ShapeDtypeStruct((epad, H), jnp.float32),
    )(hs, typ_col, norm_col, coeff_exp, basis_stack)


def _sc_scatter(msg, dst_chunks, zeros_rows, epad):
    nchunks = epad // CHUNK
    per_tile = nchunks // NUM_WORKERS

    @functools.partial(
        pl.kernel,
        out_type=jax.ShapeDtypeStruct((2, N_PAD, H), jnp.float32),
        mesh=_sc_mesh(),
        scratch_types=[
            pltpu.VMEM_SHARED((N_PAD, H), jnp.float32),
            pltpu.VMEM((CHUNK, H), jnp.float32),
            pltpu.VMEM((1, CHUNK), jnp.int32),
        ],
    )
    def scatter_kernel(msg_hbm, dst_hbm, zero_hbm, out_hbm,
                       agg_sh, msg_v, dst_v):
        cid = lax.axis_index("core")
        sid = lax.axis_index("subcore")
        wid = sid * 2 + cid

        pltpu.sync_copy(zero_hbm, agg_sh.at[pl.ds(sid * ROWS_PER_TILE, ROWS_PER_TILE)])
        plsc.subcore_barrier()

        @pl.loop(0, per_tile)
        def _(j):
            ch = wid * per_tile + j
            base = ch * CHUNK
            pltpu.sync_copy(msg_hbm.at[pl.ds(base, CHUNK)], msg_v)
            pltpu.sync_copy(dst_hbm.at[ch], dst_v.at[0])
            pltpu.sync_copy(msg_v, agg_sh.at[dst_v.at[0]], add=True)

        plsc.subcore_barrier()
        pltpu.sync_copy(
            agg_sh.at[pl.ds(sid * ROWS_PER_TILE, ROWS_PER_TILE)],
            out_hbm.at[cid, pl.ds(sid * ROWS_PER_TILE, ROWS_PER_TILE)])

    return scatter_kernel(msg, dst_chunks, zeros_rows)


def _tc_final(aggs, h, w_self, bias_row):
    nblk = N_NODES // BLK_N
    naggs = len(aggs)

    def body(*refs):
        agg_refs = refs[:naggs]
        h_ref, w_ref, b_ref, out_ref = refs[naggs:]
        acc = jnp.dot(h_ref[...], w_ref[...],
                      preferred_element_type=jnp.float32) + b_ref[...]
        for agg_ref in agg_refs:
            acc = acc + agg_ref[0] + agg_ref[1]
        out_ref[...] = acc

    return pl.pallas_call(
        body,
        grid=(nblk,),
        in_specs=(
            [pl.BlockSpec((2, BLK_N, H), lambda i: (0, i, 0))] * naggs
            + [
                pl.BlockSpec((BLK_N, H), lambda i: (i, 0)),
                pl.BlockSpec((H, H), lambda i: (0, 0)),
                pl.BlockSpec((1, H), lambda i: (0, 0)),
            ]
        ),
        out_specs=pl.BlockSpec((BLK_N, H), lambda i: (i, 0)),
        out_shape=jax.ShapeDtypeStruct((N_NODES, H), jnp.float32),
    )(*aggs, h, w_self, bias_row)


def kernel(node_id, edge_index, edge_type, edge_norm, emb_table, basis, coeff,
           w_self, bias):
    del node_id
    h = emb_table
    n_edges = edge_index.shape[1]

    grain = CHUNK * NUM_WORKERS * NCH
    epad = ((n_edges + grain - 1) // grain) * grain
    pad = epad - n_edges

    src = jnp.pad(edge_index[0].astype(jnp.int32), (0, pad))
    dst = jnp.pad(edge_index[1].astype(jnp.int32), (0, pad))
    typ = jnp.pad(edge_type.astype(jnp.int32), (0, pad))
    norm = jnp.pad(edge_norm.astype(jnp.float32), (0, pad))

    nchunks = epad // CHUNK
    src_chunks = src.reshape(nchunks, CHUNK)
    dst_chunks = dst.reshape(nchunks, CHUNK)
    typ_col = typ.reshape(epad, 1)
    norm_col = norm.reshape(epad, 1)
    esub = epad // NCH
    csub = nchunks // NCH

    basis_stack = basis.reshape(NUM_BASES * H, H).astype(jnp.bfloat16)
    coeff_exp = jnp.broadcast_to(
        coeff.astype(jnp.bfloat16)[:, :, None],
        (coeff.shape[0], NUM_BASES, H)).reshape(coeff.shape[0], NUM_BASES * H)
    bias_row = bias.reshape(1, H)
    zeros_rows = jnp.zeros((ROWS_PER_TILE, H), jnp.float32)

    aggs = []
    for c in range(NCH):
        hs_c = _sc_gather(h, src[c * esub:(c + 1) * esub], esub)
        msg_c = _tc_transform(hs_c,
                              typ_col[c * esub:(c + 1) * esub],
                              norm_col[c * esub:(c + 1) * esub],
                              coeff_exp, basis_stack, esub)
        aggs.append(_sc_scatter(msg_c,
                                dst_chunks[c * csub:(c + 1) * csub],
                                zeros_rows, esub))
    out = _tc_final([a[:, :N_NODES] for a in aggs], h, w_self, bias_row)
    return out


# speedup vs baseline: 1.2034x; 1.0162x over previous
import functools

import jax
import jax.numpy as jnp
from jax import lax
from jax.experimental import pallas as pl
from jax.experimental.pallas import tpu as pltpu
from jax.experimental.pallas import tpu_sc as plsc

N_NODES = 10000
H = 128
NUM_BASES = 16

NUM_WORKERS = 32
CHUNK = 128
BLK_E = 2048
NCH = 4
BLK_N = 1000
N_PAD = 10240
ROWS_PER_TILE = N_PAD // 16


def _sc_mesh():
    return plsc.VectorSubcoreMesh(core_axis_name="core", subcore_axis_name="subcore")


def _sc_gather(h, src_chunks, epad):
    nchunks = epad // CHUNK
    per_tile = nchunks // NUM_WORKERS

    @functools.partial(
        pl.kernel,
        out_type=jax.ShapeDtypeStruct((epad, H), jnp.float32),
        mesh=_sc_mesh(),
        scratch_types=[
            pltpu.VMEM((CHUNK,), jnp.int32),
            pltpu.VMEM((CHUNK, H), jnp.float32),
        ],
    )
    def gather_kernel(h_hbm, src_hbm, hs_hbm, src_v, hs_v):
        cid = lax.axis_index("core")
        sid = lax.axis_index("subcore")
        wid = sid * 2 + cid

        @pl.loop(0, per_tile)
        def _(j):
            base = (wid * per_tile + j) * CHUNK
            pltpu.sync_copy(src_hbm.at[pl.ds(base, CHUNK)], src_v)
            pltpu.sync_copy(h_hbm.at[src_v], hs_v)
            pltpu.sync_copy(hs_v, hs_hbm.at[pl.ds(base, CHUNK)])

    return gather_kernel(h, src_chunks)


def _tc_transform(hs, typ_col, norm_col, coeff_exp, basis_stack, epad,
                  hs_off_blk=0):
    nblk = epad // BLK_E
    num_rel = coeff_exp.shape[0]

    def body(hs_ref, typ_ref, nrm_ref, coeff_ref, basis_ref, out_ref):
        rel_iota = lax.broadcasted_iota(jnp.int32, (BLK_E, num_rel), 1)
        onehot = ((typ_ref[...] == rel_iota).astype(jnp.float32)
                  * nrm_ref[...]).astype(jnp.bfloat16)
        ce_exp = jnp.dot(onehot, coeff_ref[...],
                         preferred_element_type=jnp.float32
                         ).astype(jnp.bfloat16)
        hs_b = hs_ref[...].astype(jnp.bfloat16)
        hs_t = jnp.concatenate([hs_b] * NUM_BASES, axis=1)
        x = hs_t * ce_exp
        out_ref[...] = jnp.dot(x, basis_ref[...],
                               preferred_element_type=jnp.float32)

    return pl.pallas_call(
        body,
        grid=(nblk,),
        in_specs=[
            pl.BlockSpec((BLK_E, H), lambda i, o=hs_off_blk: (i + o, 0)),
            pl.BlockSpec((BLK_E, 1), lambda i: (i, 0)),
            pl.BlockSpec((BLK_E, 1), lambda i: (i, 0)),
            pl.BlockSpec((num_rel, NUM_BASES * H), lambda i: (0, 0)),
            pl.BlockSpec((NUM_BASES * H, H), lambda i: (0, 0)),
        ],
        out_specs=pl.BlockSpec((BLK_E, H), lambda i: (i, 0)),
        out_shape=jax.ShapeDtypeStruct((epad, H), jnp.float32),
    )(hs, typ_col, norm_col, coeff_exp, basis_stack)


def _sc_scatter(msg, dst_chunks, zeros_rows, epad):
    nchunks = epad // CHUNK
    per_tile = nchunks // NUM_WORKERS

    @functools.partial(
        pl.kernel,
        out_type=jax.ShapeDtypeStruct((2, N_PAD, H), jnp.float32),
        mesh=_sc_mesh(),
        scratch_types=[
            pltpu.VMEM_SHARED((N_PAD, H), jnp.float32),
            pltpu.VMEM((CHUNK, H), jnp.float32),
            pltpu.VMEM((1, CHUNK), jnp.int32),
        ],
    )
    def scatter_kernel(msg_hbm, dst_hbm, zero_hbm, out_hbm,
                       agg_sh, msg_v, dst_v):
        cid = lax.axis_index("core")
        sid = lax.axis_index("subcore")
        wid = sid * 2 + cid

        pltpu.sync_copy(zero_hbm, agg_sh.at[pl.ds(sid * ROWS_PER_TILE, ROWS_PER_TILE)])
        plsc.subcore_barrier()

        @pl.loop(0, per_tile)
        def _(j):
            ch = wid * per_tile + j
            base = ch * CHUNK
            pltpu.sync_copy(msg_hbm.at[pl.ds(base, CHUNK)], msg_v)
            pltpu.sync_copy(dst_hbm.at[ch], dst_v.at[0])
            pltpu.sync_copy(msg_v, agg_sh.at[dst_v.at[0]], add=True)

        plsc.subcore_barrier()
        pltpu.sync_copy(
            agg_sh.at[pl.ds(sid * ROWS_PER_TILE, ROWS_PER_TILE)],
            out_hbm.at[cid, pl.ds(sid * ROWS_PER_TILE, ROWS_PER_TILE)])

    return scatter_kernel(msg, dst_chunks, zeros_rows)


def _tc_final(aggs, h, w_self, bias_row):
    nblk = N_NODES // BLK_N
    naggs = len(aggs)

    def body(*refs):
        agg_refs = refs[:naggs]
        h_ref, w_ref, b_ref, out_ref = refs[naggs:]
        acc = jnp.dot(h_ref[...], w_ref[...],
                      preferred_element_type=jnp.float32) + b_ref[...]
        for agg_ref in agg_refs:
            acc = acc + agg_ref[0] + agg_ref[1]
        out_ref[...] = acc

    return pl.pallas_call(
        body,
        grid=(nblk,),
        in_specs=(
            [pl.BlockSpec((2, BLK_N, H), lambda i: (0, i, 0))] * naggs
            + [
                pl.BlockSpec((BLK_N, H), lambda i: (i, 0)),
                pl.BlockSpec((H, H), lambda i: (0, 0)),
                pl.BlockSpec((1, H), lambda i: (0, 0)),
            ]
        ),
        out_specs=pl.BlockSpec((BLK_N, H), lambda i: (i, 0)),
        out_shape=jax.ShapeDtypeStruct((N_NODES, H), jnp.float32),
    )(*aggs, h, w_self, bias_row)


def kernel(node_id, edge_index, edge_type, edge_norm, emb_table, basis, coeff,
           w_self, bias):
    del node_id
    h = emb_table
    n_edges = edge_index.shape[1]

    grain = CHUNK * NUM_WORKERS * NCH
    epad = ((n_edges + grain - 1) // grain) * grain
    pad = epad - n_edges

    src = jnp.pad(edge_index[0].astype(jnp.int32), (0, pad))
    dst = jnp.pad(edge_index[1].astype(jnp.int32), (0, pad))
    typ = jnp.pad(edge_type.astype(jnp.int32), (0, pad))
    norm = jnp.pad(edge_norm.astype(jnp.float32), (0, pad))

    nchunks = epad // CHUNK
    src_chunks = src.reshape(nchunks, CHUNK)
    dst_chunks = dst.reshape(nchunks, CHUNK)
    typ_col = typ.reshape(epad, 1)
    norm_col = norm.reshape(epad, 1)
    esub = epad // NCH
    csub = nchunks // NCH

    basis_stack = basis.reshape(NUM_BASES * H, H).astype(jnp.bfloat16)
    coeff_exp = jnp.broadcast_to(
        coeff.astype(jnp.bfloat16)[:, :, None],
        (coeff.shape[0], NUM_BASES, H)).reshape(coeff.shape[0], NUM_BASES * H)
    bias_row = bias.reshape(1, H)
    zeros_rows = jnp.zeros((ROWS_PER_TILE, H), jnp.float32)

    aggs = []
    for c in range(NCH):
        hs_c = _sc_gather(h, src[c * esub:(c + 1) * esub], esub)
        msg_c = _tc_transform(hs_c,
                              typ_col[c * esub:(c + 1) * esub],
                              norm_col[c * esub:(c + 1) * esub],
                              coeff_exp, basis_stack, esub)
        aggs.append(_sc_scatter(msg_c,
                                dst_chunks[c * csub:(c + 1) * csub],
                                zeros_rows, esub))
    out = _tc_final([a[:, :N_NODES] for a in aggs], h, w_self, bias_row)
    return out
